# baseline (speedup 1.0000x reference)
import jax
import jax.numpy as jnp
from jax import lax
from jax.experimental import pallas as pl
from jax.experimental.pallas import tpu as pltpu
from jax.experimental.pallas import tpu_sc as plsc

_NUM_EXPERTS = 8
_TOP_K = 2
_D = 768
_N = 8192
_CAP = 4096
_LOSS_COEF = 0.01
_TB = 512
_FT = 256
_NT = 73
_DUMP = 72 * _FT
_RBUF = _NT * _FT
_LANES = 128
_NW = 32
_TPW = _N // _NW
_CH = 64
_NCH = _TPW // _CH


def _router_kernel(x_ref, wg_ref,
                   i0_ref, i1_ref, p0_ref, p1_ref, w0_ref, w1_ref,
                   off_ref, tbl_ref, loss_ref,
                   cnt_ref, imp_ref, load_ref):
    i = pl.program_id(0)
    nblocks = pl.num_programs(0)

    @pl.when(i == 0)
    def _init():
        cnt_ref[...] = jnp.zeros_like(cnt_ref)
        imp_ref[...] = jnp.zeros_like(imp_ref)
        load_ref[...] = jnp.zeros_like(load_ref)

    xb = x_ref[...]
    wg = wg_ref[...]
    logits = jax.lax.dot_general(
        xb, wg, (((1,), (0,)), ((), ())),
        preferred_element_type=jnp.float32)
    lane = jax.lax.broadcasted_iota(jnp.int32, logits.shape, 1)
    valid = lane < _NUM_EXPERTS
    neg = jnp.float32(-1e30)
    logits = jnp.where(valid, logits, neg)

    m0 = jnp.max(logits, axis=1, keepdims=True)
    is0 = logits == m0
    idx0 = jnp.min(jnp.where(is0, lane, _LANES), axis=1, keepdims=True)
    oh0 = lane == idx0
    logits1 = jnp.where(oh0, neg, logits)
    m1 = jnp.max(logits1, axis=1, keepdims=True)
    is1 = logits1 == m1
    idx1 = jnp.min(jnp.where(is1, lane, _LANES), axis=1, keepdims=True)
    oh1 = lane == idx1

    e1 = jnp.exp(m1 - m0)
    denom = 1.0 + e1
    g0 = 1.0 / denom
    g1 = e1 / denom

    oh0f = oh0.astype(jnp.float32)
    oh1f = oh1.astype(jnp.float32)
    gates = g0 * oh0f + g1 * oh1f
    imp_ref[...] += jnp.sum(gates, axis=0, keepdims=True)
    load_ref[...] += jnp.sum((gates > 0).astype(jnp.float32), axis=0,
                             keepdims=True)

    onehot2 = oh0f + oh1f
    row = jax.lax.broadcasted_iota(jnp.int32, (_TB, _TB), 0)
    col = jax.lax.broadcasted_iota(jnp.int32, (_TB, _TB), 1)
    tri = (col < row).astype(jnp.bfloat16)
    csum = jax.lax.dot_general(
        tri, onehot2.astype(jnp.bfloat16), (((1,), (0,)), ((), ())),
        preferred_element_type=jnp.float32)
    pos_before = cnt_ref[...] + csum
    pos0 = jnp.sum(pos_before * oh0f, axis=1, keepdims=True)
    pos1 = jnp.sum(pos_before * oh1f, axis=1, keepdims=True)
    keep0 = (pos0 < _CAP).astype(jnp.float32)
    keep1 = (pos1 < _CAP).astype(jnp.float32)

    i0_ref[...] = idx0
    i1_ref[...] = idx1
    p0_ref[...] = pos0.astype(jnp.int32)
    p1_ref[...] = pos1.astype(jnp.int32)
    w0_ref[...] = g0 * keep0
    w1_ref[...] = g1 * keep1

    cnt_ref[...] += jnp.sum(onehot2, axis=0, keepdims=True)

    @pl.when(i == nblocks - 1)
    def _fin():
        inv_e = 1.0 / _NUM_EXPERTS
        lane1 = jax.lax.broadcasted_iota(jnp.int32, (1, _LANES), 1)
        vmask = (lane1 < _NUM_EXPERTS).astype(jnp.float32)

        def cv2(v):
            mean = jnp.sum(v * vmask) * inv_e
            var = jnp.sum((v - mean) ** 2 * vmask) * inv_e
            return var / (mean * mean + 1e-10)

        lv = (cv2(imp_ref[...]) + cv2(load_ref[...])) * _LOSS_COEF
        loss_ref[...] = jnp.full((1, 1), lv, jnp.float32)

        cnt = cnt_ref[...]
        kept = jnp.minimum(cnt, float(_CAP))
        padded = jnp.floor((kept + (_FT - 1)) * (1.0 / _FT)) * _FT
        rr = jax.lax.broadcasted_iota(jnp.int32, (_LANES, _LANES), 0)
        cc = jax.lax.broadcasted_iota(jnp.int32, (_LANES, _LANES), 1)
        tri2 = (rr < cc).astype(jnp.float32)
        offs = jax.lax.dot_general(
            padded, tri2, (((1,), (0,)), ((), ())),
            preferred_element_type=jnp.float32)
        off_ref[...] = offs
        base = (lane1 * _FT).astype(jnp.float32)
        acc = jnp.zeros((1, _LANES), jnp.int32)
        for e in range(_NUM_EXPERTS):
            off_e = jnp.sum(jnp.where(lane1 == e, offs, 0.0))
            acc = acc + (base >= off_e).astype(jnp.int32)
        tbl_ref[...] = jnp.clip(acc - 1, 0, _NUM_EXPERTS - 1)


def _dest_kernel(i0_ref, i1_ref, p0_ref, p1_ref, off_ref, d0_ref, d1_ref):
    offs = off_ref[...]
    lane = jax.lax.broadcasted_iota(jnp.int32, (1, _LANES), 1)

    def dest_of(idx, pos):
        off_sel = jnp.zeros_like(pos)
        for e in range(_NUM_EXPERTS):
            off_e = jnp.sum(jnp.where(lane == e, offs, 0.0)).astype(jnp.int32)
            off_sel = jnp.where(idx == e, off_e, off_sel)
        return jnp.where(pos < _CAP, off_sel + pos, _DUMP)

    d0_ref[...] = dest_of(i0_ref[...], p0_ref[...])
    d1_ref[...] = dest_of(i1_ref[...], p1_ref[...])


def _sc_dispatch_body(x_hbm, d0_hbm, d1_hbm, buf_hbm,
                      d0_v, d1_v, rows_a, rows_b, si_a, si_b, so_a, so_b):
    c = lax.axis_index("c")
    s = lax.axis_index("s")
    wid = s * 2 + c
    base = wid * _TPW

    pltpu.sync_copy(d0_hbm.at[pl.ds(wid * _NCH, _NCH)], d0_v)
    pltpu.sync_copy(d1_hbm.at[pl.ds(wid * _NCH, _NCH)], d1_v)

    rows = [rows_a, rows_b]
    si = [si_a, si_b]
    so = [so_a, so_b]

    def load(ch, b):
        return pltpu.async_copy(
            x_hbm.at[pl.ds(base + ch * _CH, _CH)], rows[b], si[b])

    ins = {0: load(0, 0), 1: load(1, 1)}
    outs = {}
    for ch in range(_NCH):
        b = ch & 1
        ins[ch].wait()
        s0 = pltpu.async_copy(rows[b], buf_hbm.at[d0_v.at[ch]], so[b])
        s1 = pltpu.async_copy(rows[b], buf_hbm.at[d1_v.at[ch]], so[b])
        outs[ch] = (s0, s1)
        if ch + 2 < _NCH:
            s0.wait()
            s1.wait()
            ins[ch + 2] = load(ch + 2, b)
    for ch in range(max(0, _NCH - 2), _NCH):
        outs[ch][0].wait()
        outs[ch][1].wait()


def _sc_combine_body(y_hbm, d0_hbm, d1_hbm, a0_hbm, a1_hbm,
                     d0_v, d1_v, rows_a, rows_b, si_a, si_b, so_a, so_b):
    c = lax.axis_index("c")
    s = lax.axis_index("s")
    wid = s * 2 + c
    base = wid * _TPW

    pltpu.sync_copy(d0_hbm.at[pl.ds(wid * _NCH, _NCH)], d0_v)
    pltpu.sync_copy(d1_hbm.at[pl.ds(wid * _NCH, _NCH)], d1_v)

    rows = [rows_a, rows_b]
    si = [si_a, si_b]
    so = [so_a, so_b]
    dv = [d0_v, d1_v]
    ah = [a0_hbm, a1_hbm]
    ntask = 2 * _NCH

    def gather(k, b):
        slot, ch = k & 1, k >> 1
        return pltpu.async_copy(y_hbm.at[dv[slot].at[ch]], rows[b], si[b])

    ins = {0: gather(0, 0), 1: gather(1, 1)}
    outs = {}
    for k in range(ntask):
        b = k & 1
        slot, ch = k & 1, k >> 1
        ins[k].wait()
        w = pltpu.async_copy(
            rows[b], ah[slot].at[pl.ds(base + ch * _CH, _CH)], so[b])
        outs[k] = w
        if k + 2 < ntask:
            w.wait()
            ins[k + 2] = gather(k + 2, b)
    for k in range(max(0, ntask - 2), ntask):
        outs[k].wait()


def _ffn_kernel(tbl_ref, buf_ref, w1_ref, b1_ref, w2_ref, b2_ref, y_ref):
    hf = _FT // 2

    def dot(a, b):
        return jax.lax.dot_general(
            a, b, (((1,), (0,)), ((), ())),
            preferred_element_type=jnp.float32)

    xa = buf_ref[:hf, :].astype(jnp.bfloat16)
    xb = buf_ref[hf:, :].astype(jnp.bfloat16)
    w1 = w1_ref[0]
    w2 = w2_ref[0]
    ha = dot(xa, w1) + b1_ref[0]
    hb = dot(xb, w1) + b1_ref[0]
    ga = jax.nn.gelu(ha.astype(jnp.bfloat16))
    gb = jax.nn.gelu(hb.astype(jnp.bfloat16))
    y_ref[:hf, :] = dot(ga, w2) + b2_ref[0]
    y_ref[hf:, :] = dot(gb, w2) + b2_ref[0]


def _combine_kernel(x_ref, a0_ref, a1_ref, w0_ref, w1_ref, out_ref):
    w0 = w0_ref[...]
    w1 = w1_ref[...]
    out_ref[...] = (x_ref[...]
                    + jnp.where(w0 > 0, w0 * a0_ref[...], 0.0)
                    + jnp.where(w1 > 0, w1 * a1_ref[...], 0.0))


def _run_router(x, w_gate):
    wg_pad = jnp.zeros((_D, _LANES), jnp.float32).at[:, :_NUM_EXPERTS].set(
        w_gate)
    nb = _N // _TB
    return pl.pallas_call(
        _router_kernel,
        grid=(nb,),
        in_specs=[
            pl.BlockSpec((_TB, _D), lambda i: (i, 0)),
            pl.BlockSpec((_D, _LANES), lambda i: (0, 0)),
        ],
        out_specs=[
            pl.BlockSpec((_TB, 1), lambda i: (i, 0)),
            pl.BlockSpec((_TB, 1), lambda i: (i, 0)),
            pl.BlockSpec((_TB, 1), lambda i: (i, 0)),
            pl.BlockSpec((_TB, 1), lambda i: (i, 0)),
            pl.BlockSpec((_TB, 1), lambda i: (i, 0)),
            pl.BlockSpec((_TB, 1), lambda i: (i, 0)),
            pl.BlockSpec((1, _LANES), lambda i: (0, 0)),
            pl.BlockSpec((1, _LANES), lambda i: (0, 0)),
            pl.BlockSpec((1, 1), lambda i: (0, 0)),
        ],
        out_shape=[
            jax.ShapeDtypeStruct((_N, 1), jnp.int32),
            jax.ShapeDtypeStruct((_N, 1), jnp.int32),
            jax.ShapeDtypeStruct((_N, 1), jnp.int32),
            jax.ShapeDtypeStruct((_N, 1), jnp.int32),
            jax.ShapeDtypeStruct((_N, 1), jnp.float32),
            jax.ShapeDtypeStruct((_N, 1), jnp.float32),
            jax.ShapeDtypeStruct((1, _LANES), jnp.float32),
            jax.ShapeDtypeStruct((1, _LANES), jnp.int32),
            jax.ShapeDtypeStruct((1, 1), jnp.float32),
        ],
        scratch_shapes=[
            pltpu.VMEM((1, _LANES), jnp.float32),
            pltpu.VMEM((1, _LANES), jnp.float32),
            pltpu.VMEM((1, _LANES), jnp.float32),
        ],
    )(x, wg_pad)


def _run_dest(i0, i1, p0, p1, offs):
    nb = _N // _TB
    return pl.pallas_call(
        _dest_kernel,
        grid=(nb,),
        in_specs=[
            pl.BlockSpec((_TB, 1), lambda i: (i, 0)),
            pl.BlockSpec((_TB, 1), lambda i: (i, 0)),
            pl.BlockSpec((_TB, 1), lambda i: (i, 0)),
            pl.BlockSpec((_TB, 1), lambda i: (i, 0)),
            pl.BlockSpec((1, _LANES), lambda i: (0, 0)),
        ],
        out_specs=[
            pl.BlockSpec((_TB, 1), lambda i: (i, 0)),
            pl.BlockSpec((_TB, 1), lambda i: (i, 0)),
        ],
        out_shape=[
            jax.ShapeDtypeStruct((_N, 1), jnp.int32),
            jax.ShapeDtypeStruct((_N, 1), jnp.int32),
        ],
    )(i0, i1, p0, p1, offs)


_SC_SCRATCH = [
    pltpu.VMEM((_NCH, _CH), jnp.int32),
    pltpu.VMEM((_NCH, _CH), jnp.int32),
    pltpu.VMEM((_CH, _D), jnp.float32),
    pltpu.VMEM((_CH, _D), jnp.float32),
    pltpu.SemaphoreType.DMA,
    pltpu.SemaphoreType.DMA,
    pltpu.SemaphoreType.DMA,
    pltpu.SemaphoreType.DMA,
]


def _run_sc_dispatch(x, d0, d1):
    mesh = plsc.VectorSubcoreMesh(core_axis_name="c", subcore_axis_name="s")
    kfn = pl.kernel(
        _sc_dispatch_body,
        mesh=mesh,
        out_type=jax.ShapeDtypeStruct((_RBUF, _D), jnp.float32),
        scratch_types=_SC_SCRATCH,
    )
    return kfn(x, d0, d1)


def _run_sc_combine(y, d0, d1):
    mesh = plsc.VectorSubcoreMesh(core_axis_name="c", subcore_axis_name="s")
    kfn = pl.kernel(
        _sc_combine_body,
        mesh=mesh,
        out_type=[
            jax.ShapeDtypeStruct((_N, _D), jnp.float32),
            jax.ShapeDtypeStruct((_N, _D), jnp.float32),
        ],
        scratch_types=_SC_SCRATCH,
    )
    return kfn(y, d0, d1)


def _run_combine(x, a0, a1, w0, w1):
    return pl.pallas_call(
        _combine_kernel,
        grid=(_N // _TB,),
        in_specs=[
            pl.BlockSpec((_TB, _D), lambda i: (i, 0)),
            pl.BlockSpec((_TB, _D), lambda i: (i, 0)),
            pl.BlockSpec((_TB, _D), lambda i: (i, 0)),
            pl.BlockSpec((_TB, 1), lambda i: (i, 0)),
            pl.BlockSpec((_TB, 1), lambda i: (i, 0)),
        ],
        out_specs=pl.BlockSpec((_TB, _D), lambda i: (i, 0)),
        out_shape=jax.ShapeDtypeStruct((_N, _D), jnp.float32),
    )(x, a0, a1, w0, w1)


def _run_ffn(buf, tbl, W1h, b1, W2h, b2):
    return pl.pallas_call(
        _ffn_kernel,
        grid_spec=pltpu.PrefetchScalarGridSpec(
            num_scalar_prefetch=1,
            grid=(_NT - 1,),
            in_specs=[
                pl.BlockSpec((_FT, _D), lambda i, tbl: (i, 0)),
                pl.BlockSpec((1, _D, _D), lambda i, tbl: (tbl[i], 0, 0)),
                pl.BlockSpec((1, 1, _D), lambda i, tbl: (tbl[i], 0, 0)),
                pl.BlockSpec((1, _D, _D), lambda i, tbl: (tbl[i], 0, 0)),
                pl.BlockSpec((1, 1, _D), lambda i, tbl: (tbl[i], 0, 0)),
            ],
            out_specs=pl.BlockSpec((_FT, _D), lambda i, tbl: (i, 0)),
        ),
        out_shape=jax.ShapeDtypeStruct((_RBUF, _D), jnp.float32),
    )(tbl, buf, W1h, b1, W2h, b2)


@jax.jit
def kernel(x, w_gate, W1, b1, W2, b2):
    i0, i1, p0, p1, w0, w1, offs, tbl, loss = _run_router(x, w_gate)
    d0, d1 = _run_dest(i0, i1, p0, p1, offs)

    d0r = jnp.reshape(d0, (_N // _CH, _CH))
    d1r = jnp.reshape(d1, (_N // _CH, _CH))
    tbl_flat = jnp.reshape(tbl, (_LANES,))

    buf = _run_sc_dispatch(x, d0r, d1r)
    y = _run_ffn(buf, tbl_flat, W1.astype(jnp.bfloat16),
                 b1.reshape(_NUM_EXPERTS, 1, _D),
                 W2.astype(jnp.bfloat16),
                 b2.reshape(_NUM_EXPERTS, 1, _D))
    a0, a1 = _run_sc_combine(y, d0r, d1r)
    out = _run_combine(x, a0, a1, w0, w1)
    return (out, jnp.reshape(loss, ()))


# speedup vs baseline: 1.0359x; 1.0359x over previous
import jax
import jax.numpy as jnp
from jax import lax
from jax.experimental import pallas as pl
from jax.experimental.pallas import tpu as pltpu
from jax.experimental.pallas import tpu_sc as plsc

_NUM_EXPERTS = 8
_TOP_K = 2
_D = 768
_N = 8192
_CAP = 4096
_LOSS_COEF = 0.01
_TB = 1024
_FT = 256
_NT = 73
_DUMP = 72 * _FT
_RBUF = _NT * _FT
_LANES = 128
_NW = 32
_TPW = _N // _NW
_CH = 64
_NCH = _TPW // _CH


def _router_kernel(x_ref, wg_ref,
                   i0_ref, i1_ref, p0_ref, p1_ref, w0_ref, w1_ref,
                   off_ref, tbl_ref, loss_ref,
                   cnt_ref, imp_ref, load_ref):
    i = pl.program_id(0)
    nblocks = pl.num_programs(0)

    @pl.when(i == 0)
    def _init():
        cnt_ref[...] = jnp.zeros_like(cnt_ref)
        imp_ref[...] = jnp.zeros_like(imp_ref)
        load_ref[...] = jnp.zeros_like(load_ref)

    xb = x_ref[...]
    wg = wg_ref[...]
    logits = jax.lax.dot_general(
        xb, wg, (((1,), (0,)), ((), ())),
        preferred_element_type=jnp.float32)
    lane = jax.lax.broadcasted_iota(jnp.int32, logits.shape, 1)
    valid = lane < _NUM_EXPERTS
    neg = jnp.float32(-1e30)
    logits = jnp.where(valid, logits, neg)

    m0 = jnp.max(logits, axis=1, keepdims=True)
    is0 = logits == m0
    idx0 = jnp.min(jnp.where(is0, lane, _LANES), axis=1, keepdims=True)
    oh0 = lane == idx0
    logits1 = jnp.where(oh0, neg, logits)
    m1 = jnp.max(logits1, axis=1, keepdims=True)
    is1 = logits1 == m1
    idx1 = jnp.min(jnp.where(is1, lane, _LANES), axis=1, keepdims=True)
    oh1 = lane == idx1

    e1 = jnp.exp(m1 - m0)
    denom = 1.0 + e1
    g0 = 1.0 / denom
    g1 = e1 / denom

    oh0f = oh0.astype(jnp.float32)
    oh1f = oh1.astype(jnp.float32)
    gates = g0 * oh0f + g1 * oh1f
    imp_ref[...] += jnp.sum(gates, axis=0, keepdims=True)
    load_ref[...] += jnp.sum((gates > 0).astype(jnp.float32), axis=0,
                             keepdims=True)

    onehot2 = oh0f + oh1f
    row = jax.lax.broadcasted_iota(jnp.int32, (_TB, _TB), 0)
    col = jax.lax.broadcasted_iota(jnp.int32, (_TB, _TB), 1)
    tri = (col < row).astype(jnp.bfloat16)
    csum = jax.lax.dot_general(
        tri, onehot2.astype(jnp.bfloat16), (((1,), (0,)), ((), ())),
        preferred_element_type=jnp.float32)
    pos_before = cnt_ref[...] + csum
    pos0 = jnp.sum(pos_before * oh0f, axis=1, keepdims=True)
    pos1 = jnp.sum(pos_before * oh1f, axis=1, keepdims=True)
    keep0 = (pos0 < _CAP).astype(jnp.float32)
    keep1 = (pos1 < _CAP).astype(jnp.float32)

    i0_ref[...] = idx0
    i1_ref[...] = idx1
    p0_ref[...] = pos0.astype(jnp.int32)
    p1_ref[...] = pos1.astype(jnp.int32)
    w0_ref[...] = g0 * keep0
    w1_ref[...] = g1 * keep1

    cnt_ref[...] += jnp.sum(onehot2, axis=0, keepdims=True)

    @pl.when(i == nblocks - 1)
    def _fin():
        inv_e = 1.0 / _NUM_EXPERTS
        lane1 = jax.lax.broadcasted_iota(jnp.int32, (1, _LANES), 1)
        vmask = (lane1 < _NUM_EXPERTS).astype(jnp.float32)

        def cv2(v):
            mean = jnp.sum(v * vmask) * inv_e
            var = jnp.sum((v - mean) ** 2 * vmask) * inv_e
            return var / (mean * mean + 1e-10)

        lv = (cv2(imp_ref[...]) + cv2(load_ref[...])) * _LOSS_COEF
        loss_ref[...] = jnp.full((1, 1), lv, jnp.float32)

        cnt = cnt_ref[...]
        kept = jnp.minimum(cnt, float(_CAP))
        padded = jnp.floor((kept + (_FT - 1)) * (1.0 / _FT)) * _FT
        rr = jax.lax.broadcasted_iota(jnp.int32, (_LANES, _LANES), 0)
        cc = jax.lax.broadcasted_iota(jnp.int32, (_LANES, _LANES), 1)
        tri2 = (rr < cc).astype(jnp.float32)
        offs = jax.lax.dot_general(
            padded, tri2, (((1,), (0,)), ((), ())),
            preferred_element_type=jnp.float32)
        off_ref[...] = offs
        base = (lane1 * _FT).astype(jnp.float32)
        acc = jnp.zeros((1, _LANES), jnp.int32)
        for e in range(_NUM_EXPERTS):
            off_e = jnp.sum(jnp.where(lane1 == e, offs, 0.0))
            acc = acc + (base >= off_e).astype(jnp.int32)
        tbl_ref[...] = jnp.clip(acc - 1, 0, _NUM_EXPERTS - 1)


def _dest_kernel(i0_ref, i1_ref, p0_ref, p1_ref, off_ref, d0_ref, d1_ref):
    offs = off_ref[...]
    lane = jax.lax.broadcasted_iota(jnp.int32, (1, _LANES), 1)

    def dest_of(idx, pos):
        off_sel = jnp.zeros_like(pos)
        for e in range(_NUM_EXPERTS):
            off_e = jnp.sum(jnp.where(lane == e, offs, 0.0)).astype(jnp.int32)
            off_sel = jnp.where(idx == e, off_e, off_sel)
        return jnp.where(pos < _CAP, off_sel + pos, _DUMP)

    d0_ref[...] = dest_of(i0_ref[...], p0_ref[...])
    d1_ref[...] = dest_of(i1_ref[...], p1_ref[...])


def _sc_dispatch_body(x_hbm, d0_hbm, d1_hbm, buf_hbm,
                      d0_v, d1_v, rows_a, rows_b, si_a, si_b, so_a, so_b):
    c = lax.axis_index("c")
    s = lax.axis_index("s")
    wid = s * 2 + c
    base = wid * _TPW

    pltpu.sync_copy(d0_hbm.at[pl.ds(wid * _NCH, _NCH)], d0_v)
    pltpu.sync_copy(d1_hbm.at[pl.ds(wid * _NCH, _NCH)], d1_v)

    rows = [rows_a, rows_b]
    si = [si_a, si_b]
    so = [so_a, so_b]

    def load(ch, b):
        return pltpu.async_copy(
            x_hbm.at[pl.ds(base + ch * _CH, _CH)], rows[b], si[b])

    ins = {0: load(0, 0), 1: load(1, 1)}
    outs = {}
    for ch in range(_NCH):
        b = ch & 1
        ins[ch].wait()
        s0 = pltpu.async_copy(rows[b], buf_hbm.at[d0_v.at[ch]], so[b])
        s1 = pltpu.async_copy(rows[b], buf_hbm.at[d1_v.at[ch]], so[b])
        outs[ch] = (s0, s1)
        if ch + 2 < _NCH:
            s0.wait()
            s1.wait()
            ins[ch + 2] = load(ch + 2, b)
    for ch in range(max(0, _NCH - 2), _NCH):
        outs[ch][0].wait()
        outs[ch][1].wait()


def _sc_combine_body(y_hbm, d0_hbm, d1_hbm, a0_hbm, a1_hbm,
                     d0_v, d1_v, rows_a, rows_b, si_a, si_b, so_a, so_b):
    c = lax.axis_index("c")
    s = lax.axis_index("s")
    wid = s * 2 + c
    base = wid * _TPW

    pltpu.sync_copy(d0_hbm.at[pl.ds(wid * _NCH, _NCH)], d0_v)
    pltpu.sync_copy(d1_hbm.at[pl.ds(wid * _NCH, _NCH)], d1_v)

    rows = [rows_a, rows_b]
    si = [si_a, si_b]
    so = [so_a, so_b]
    dv = [d0_v, d1_v]
    ah = [a0_hbm, a1_hbm]
    ntask = 2 * _NCH

    def gather(k, b):
        slot, ch = k & 1, k >> 1
        return pltpu.async_copy(y_hbm.at[dv[slot].at[ch]], rows[b], si[b])

    ins = {0: gather(0, 0), 1: gather(1, 1)}
    outs = {}
    for k in range(ntask):
        b = k & 1
        slot, ch = k & 1, k >> 1
        ins[k].wait()
        w = pltpu.async_copy(
            rows[b], ah[slot].at[pl.ds(base + ch * _CH, _CH)], so[b])
        outs[k] = w
        if k + 2 < ntask:
            w.wait()
            ins[k + 2] = gather(k + 2, b)
    for k in range(max(0, ntask - 2), ntask):
        outs[k].wait()


def _ffn_kernel(tbl_ref, buf_ref, w1_ref, b1_ref, w2_ref, b2_ref, y_ref):
    hf = _FT // 2

    def dot(a, b):
        return jax.lax.dot_general(
            a, b, (((1,), (0,)), ((), ())),
            preferred_element_type=jnp.float32)

    xa = buf_ref[:hf, :].astype(jnp.bfloat16)
    xb = buf_ref[hf:, :].astype(jnp.bfloat16)
    w1 = w1_ref[0]
    w2 = w2_ref[0]
    ha = dot(xa, w1) + b1_ref[0]
    hb = dot(xb, w1) + b1_ref[0]
    ga = jax.nn.gelu(ha.astype(jnp.bfloat16))
    gb = jax.nn.gelu(hb.astype(jnp.bfloat16))
    y_ref[:hf, :] = dot(ga, w2) + b2_ref[0]
    y_ref[hf:, :] = dot(gb, w2) + b2_ref[0]


def _combine_kernel(x_ref, a0_ref, a1_ref, w0_ref, w1_ref, out_ref):
    w0 = w0_ref[...]
    w1 = w1_ref[...]
    out_ref[...] = (x_ref[...]
                    + jnp.where(w0 > 0, w0 * a0_ref[...], 0.0)
                    + jnp.where(w1 > 0, w1 * a1_ref[...], 0.0))


def _run_router(x, w_gate):
    wg_pad = jnp.zeros((_D, _LANES), jnp.float32).at[:, :_NUM_EXPERTS].set(
        w_gate)
    nb = _N // _TB
    return pl.pallas_call(
        _router_kernel,
        grid=(nb,),
        in_specs=[
            pl.BlockSpec((_TB, _D), lambda i: (i, 0)),
            pl.BlockSpec((_D, _LANES), lambda i: (0, 0)),
        ],
        out_specs=[
            pl.BlockSpec((_TB, 1), lambda i: (i, 0)),
            pl.BlockSpec((_TB, 1), lambda i: (i, 0)),
            pl.BlockSpec((_TB, 1), lambda i: (i, 0)),
            pl.BlockSpec((_TB, 1), lambda i: (i, 0)),
            pl.BlockSpec((_TB, 1), lambda i: (i, 0)),
            pl.BlockSpec((_TB, 1), lambda i: (i, 0)),
            pl.BlockSpec((1, _LANES), lambda i: (0, 0)),
            pl.BlockSpec((1, _LANES), lambda i: (0, 0)),
            pl.BlockSpec((1, 1), lambda i: (0, 0)),
        ],
        out_shape=[
            jax.ShapeDtypeStruct((_N, 1), jnp.int32),
            jax.ShapeDtypeStruct((_N, 1), jnp.int32),
            jax.ShapeDtypeStruct((_N, 1), jnp.int32),
            jax.ShapeDtypeStruct((_N, 1), jnp.int32),
            jax.ShapeDtypeStruct((_N, 1), jnp.float32),
            jax.ShapeDtypeStruct((_N, 1), jnp.float32),
            jax.ShapeDtypeStruct((1, _LANES), jnp.float32),
            jax.ShapeDtypeStruct((1, _LANES), jnp.int32),
            jax.ShapeDtypeStruct((1, 1), jnp.float32),
        ],
        scratch_shapes=[
            pltpu.VMEM((1, _LANES), jnp.float32),
            pltpu.VMEM((1, _LANES), jnp.float32),
            pltpu.VMEM((1, _LANES), jnp.float32),
        ],
    )(x, wg_pad)


def _run_dest(i0, i1, p0, p1, offs):
    nb = _N // _TB
    return pl.pallas_call(
        _dest_kernel,
        grid=(nb,),
        in_specs=[
            pl.BlockSpec((_TB, 1), lambda i: (i, 0)),
            pl.BlockSpec((_TB, 1), lambda i: (i, 0)),
            pl.BlockSpec((_TB, 1), lambda i: (i, 0)),
            pl.BlockSpec((_TB, 1), lambda i: (i, 0)),
            pl.BlockSpec((1, _LANES), lambda i: (0, 0)),
        ],
        out_specs=[
            pl.BlockSpec((_TB, 1), lambda i: (i, 0)),
            pl.BlockSpec((_TB, 1), lambda i: (i, 0)),
        ],
        out_shape=[
            jax.ShapeDtypeStruct((_N, 1), jnp.int32),
            jax.ShapeDtypeStruct((_N, 1), jnp.int32),
        ],
    )(i0, i1, p0, p1, offs)


_SC_SCRATCH = [
    pltpu.VMEM((_NCH, _CH), jnp.int32),
    pltpu.VMEM((_NCH, _CH), jnp.int32),
    pltpu.VMEM((_CH, _D), jnp.float32),
    pltpu.VMEM((_CH, _D), jnp.float32),
    pltpu.SemaphoreType.DMA,
    pltpu.SemaphoreType.DMA,
    pltpu.SemaphoreType.DMA,
    pltpu.SemaphoreType.DMA,
]


def _run_sc_dispatch(x, d0, d1):
    mesh = plsc.VectorSubcoreMesh(core_axis_name="c", subcore_axis_name="s")
    kfn = pl.kernel(
        _sc_dispatch_body,
        mesh=mesh,
        out_type=jax.ShapeDtypeStruct((_RBUF, _D), jnp.float32),
        scratch_types=_SC_SCRATCH,
    )
    return kfn(x, d0, d1)


def _run_sc_combine(y, d0, d1):
    mesh = plsc.VectorSubcoreMesh(core_axis_name="c", subcore_axis_name="s")
    kfn = pl.kernel(
        _sc_combine_body,
        mesh=mesh,
        out_type=[
            jax.ShapeDtypeStruct((_N, _D), jnp.float32),
            jax.ShapeDtypeStruct((_N, _D), jnp.float32),
        ],
        scratch_types=_SC_SCRATCH,
    )
    return kfn(y, d0, d1)


def _run_combine(x, a0, a1, w0, w1):
    return pl.pallas_call(
        _combine_kernel,
        grid=(_N // _TB,),
        in_specs=[
            pl.BlockSpec((_TB, _D), lambda i: (i, 0)),
            pl.BlockSpec((_TB, _D), lambda i: (i, 0)),
            pl.BlockSpec((_TB, _D), lambda i: (i, 0)),
            pl.BlockSpec((_TB, 1), lambda i: (i, 0)),
            pl.BlockSpec((_TB, 1), lambda i: (i, 0)),
        ],
        out_specs=pl.BlockSpec((_TB, _D), lambda i: (i, 0)),
        out_shape=jax.ShapeDtypeStruct((_N, _D), jnp.float32),
    )(x, a0, a1, w0, w1)


def _run_ffn(buf, tbl, W1h, b1, W2h, b2):
    return pl.pallas_call(
        _ffn_kernel,
        grid_spec=pltpu.PrefetchScalarGridSpec(
            num_scalar_prefetch=1,
            grid=(_NT - 1,),
            in_specs=[
                pl.BlockSpec((_FT, _D), lambda i, tbl: (i, 0)),
                pl.BlockSpec((1, _D, _D), lambda i, tbl: (tbl[i], 0, 0)),
                pl.BlockSpec((1, 1, _D), lambda i, tbl: (tbl[i], 0, 0)),
                pl.BlockSpec((1, _D, _D), lambda i, tbl: (tbl[i], 0, 0)),
                pl.BlockSpec((1, 1, _D), lambda i, tbl: (tbl[i], 0, 0)),
            ],
            out_specs=pl.BlockSpec((_FT, _D), lambda i, tbl: (i, 0)),
        ),
        out_shape=jax.ShapeDtypeStruct((_RBUF, _D), jnp.float32),
    )(tbl, buf, W1h, b1, W2h, b2)


@jax.jit
def kernel(x, w_gate, W1, b1, W2, b2):
    i0, i1, p0, p1, w0, w1, offs, tbl, loss = _run_router(x, w_gate)
    d0, d1 = _run_dest(i0, i1, p0, p1, offs)

    d0r = jnp.reshape(d0, (_N // _CH, _CH))
    d1r = jnp.reshape(d1, (_N // _CH, _CH))
    tbl_flat = jnp.reshape(tbl, (_LANES,))

    buf = _run_sc_dispatch(x, d0r, d1r)
    y = _run_ffn(buf, tbl_flat, W1.astype(jnp.bfloat16),
                 b1.reshape(_NUM_EXPERTS, 1, _D),
                 W2.astype(jnp.bfloat16),
                 b2.reshape(_NUM_EXPERTS, 1, _D))
    a0, a1 = _run_sc_combine(y, d0r, d1r)
    out = _run_combine(x, a0, a1, w0, w1)
    return (out, jnp.reshape(loss, ()))


# speedup vs baseline: 1.0389x; 1.0029x over previous
import jax
import jax.numpy as jnp
from jax import lax
from jax.experimental import pallas as pl
from jax.experimental.pallas import tpu as pltpu
from jax.experimental.pallas import tpu_sc as plsc

_NUM_EXPERTS = 8
_TOP_K = 2
_D = 768
_N = 8192
_CAP = 4096
_LOSS_COEF = 0.01
_TB = 1024
_FT = 256
_NT = 73
_DUMP = 72 * _FT
_RBUF = _NT * _FT
_LANES = 128
_NW = 32
_TPW = _N // _NW
_CH = 32
_NCH = _TPW // _CH
_NB = 4


def _router_kernel(x_ref, wg_ref,
                   i0_ref, i1_ref, p0_ref, p1_ref, w0_ref, w1_ref,
                   off_ref, tbl_ref, loss_ref,
                   cnt_ref, imp_ref, load_ref):
    i = pl.program_id(0)
    nblocks = pl.num_programs(0)

    @pl.when(i == 0)
    def _init():
        cnt_ref[...] = jnp.zeros_like(cnt_ref)
        imp_ref[...] = jnp.zeros_like(imp_ref)
        load_ref[...] = jnp.zeros_like(load_ref)

    xb = x_ref[...]
    wg = wg_ref[...]
    logits = jax.lax.dot_general(
        xb, wg, (((1,), (0,)), ((), ())),
        preferred_element_type=jnp.float32)
    lane = jax.lax.broadcasted_iota(jnp.int32, logits.shape, 1)
    valid = lane < _NUM_EXPERTS
    neg = jnp.float32(-1e30)
    logits = jnp.where(valid, logits, neg)

    m0 = jnp.max(logits, axis=1, keepdims=True)
    is0 = logits == m0
    idx0 = jnp.min(jnp.where(is0, lane, _LANES), axis=1, keepdims=True)
    oh0 = lane == idx0
    logits1 = jnp.where(oh0, neg, logits)
    m1 = jnp.max(logits1, axis=1, keepdims=True)
    is1 = logits1 == m1
    idx1 = jnp.min(jnp.where(is1, lane, _LANES), axis=1, keepdims=True)
    oh1 = lane == idx1

    e1 = jnp.exp(m1 - m0)
    denom = 1.0 + e1
    g0 = 1.0 / denom
    g1 = e1 / denom

    oh0f = oh0.astype(jnp.float32)
    oh1f = oh1.astype(jnp.float32)
    gates = g0 * oh0f + g1 * oh1f
    imp_ref[...] += jnp.sum(gates, axis=0, keepdims=True)
    load_ref[...] += jnp.sum((gates > 0).astype(jnp.float32), axis=0,
                             keepdims=True)

    onehot2 = oh0f + oh1f
    row = jax.lax.broadcasted_iota(jnp.int32, (_TB, _TB), 0)
    col = jax.lax.broadcasted_iota(jnp.int32, (_TB, _TB), 1)
    tri = (col < row).astype(jnp.bfloat16)
    csum = jax.lax.dot_general(
        tri, onehot2.astype(jnp.bfloat16), (((1,), (0,)), ((), ())),
        preferred_element_type=jnp.float32)
    pos_before = cnt_ref[...] + csum
    pos0 = jnp.sum(pos_before * oh0f, axis=1, keepdims=True)
    pos1 = jnp.sum(pos_before * oh1f, axis=1, keepdims=True)
    keep0 = (pos0 < _CAP).astype(jnp.float32)
    keep1 = (pos1 < _CAP).astype(jnp.float32)

    i0_ref[...] = idx0
    i1_ref[...] = idx1
    p0_ref[...] = pos0.astype(jnp.int32)
    p1_ref[...] = pos1.astype(jnp.int32)
    w0_ref[...] = g0 * keep0
    w1_ref[...] = g1 * keep1

    cnt_ref[...] += jnp.sum(onehot2, axis=0, keepdims=True)

    @pl.when(i == nblocks - 1)
    def _fin():
        inv_e = 1.0 / _NUM_EXPERTS
        lane1 = jax.lax.broadcasted_iota(jnp.int32, (1, _LANES), 1)
        vmask = (lane1 < _NUM_EXPERTS).astype(jnp.float32)

        def cv2(v):
            mean = jnp.sum(v * vmask) * inv_e
            var = jnp.sum((v - mean) ** 2 * vmask) * inv_e
            return var / (mean * mean + 1e-10)

        lv = (cv2(imp_ref[...]) + cv2(load_ref[...])) * _LOSS_COEF
        loss_ref[...] = jnp.full((1, 1), lv, jnp.float32)

        cnt = cnt_ref[...]
        kept = jnp.minimum(cnt, float(_CAP))
        padded = jnp.floor((kept + (_FT - 1)) * (1.0 / _FT)) * _FT
        rr = jax.lax.broadcasted_iota(jnp.int32, (_LANES, _LANES), 0)
        cc = jax.lax.broadcasted_iota(jnp.int32, (_LANES, _LANES), 1)
        tri2 = (rr < cc).astype(jnp.float32)
        offs = jax.lax.dot_general(
            padded, tri2, (((1,), (0,)), ((), ())),
            preferred_element_type=jnp.float32)
        off_ref[...] = offs
        base = (lane1 * _FT).astype(jnp.float32)
        acc = jnp.zeros((1, _LANES), jnp.int32)
        for e in range(_NUM_EXPERTS):
            off_e = jnp.sum(jnp.where(lane1 == e, offs, 0.0))
            acc = acc + (base >= off_e).astype(jnp.int32)
        tbl_ref[...] = jnp.clip(acc - 1, 0, _NUM_EXPERTS - 1)


def _dest_kernel(i0_ref, i1_ref, p0_ref, p1_ref, off_ref, d0_ref, d1_ref):
    offs = off_ref[...]
    lane = jax.lax.broadcasted_iota(jnp.int32, (1, _LANES), 1)

    def dest_of(idx, pos):
        off_sel = jnp.zeros_like(pos)
        for e in range(_NUM_EXPERTS):
            off_e = jnp.sum(jnp.where(lane == e, offs, 0.0)).astype(jnp.int32)
            off_sel = jnp.where(idx == e, off_e, off_sel)
        return jnp.where(pos < _CAP, off_sel + pos, _DUMP)

    d0_ref[...] = dest_of(i0_ref[...], p0_ref[...])
    d1_ref[...] = dest_of(i1_ref[...], p1_ref[...])


def _sc_dispatch_body(x_hbm, d0_hbm, d1_hbm, buf_hbm,
                      d0_v, d1_v, r0, r1, r2, r3,
                      si0, si1, si2, si3, so0, so1, so2, so3):
    c = lax.axis_index("c")
    s = lax.axis_index("s")
    wid = s * 2 + c
    base = wid * _TPW

    pltpu.sync_copy(d0_hbm.at[pl.ds(wid * _NCH, _NCH)], d0_v)
    pltpu.sync_copy(d1_hbm.at[pl.ds(wid * _NCH, _NCH)], d1_v)

    rows = [r0, r1, r2, r3]
    si = [si0, si1, si2, si3]
    so = [so0, so1, so2, so3]

    def load(ch):
        b = ch % _NB
        return pltpu.async_copy(
            x_hbm.at[pl.ds(base + ch * _CH, _CH)], rows[b], si[b])

    ins = {ch: load(ch) for ch in range(_NB)}
    outs = {}
    for ch in range(_NCH):
        b = ch % _NB
        if ch >= 2 and ch + 2 < _NCH:
            outs[ch - 2][0].wait()
            outs[ch - 2][1].wait()
            ins[ch + 2] = load(ch + 2)
        ins[ch].wait()
        outs[ch] = (
            pltpu.async_copy(rows[b], buf_hbm.at[d0_v.at[ch]], so[b]),
            pltpu.async_copy(rows[b], buf_hbm.at[d1_v.at[ch]], so[b]),
        )
    for ch in range(max(0, _NCH - 4), _NCH):
        outs[ch][0].wait()
        outs[ch][1].wait()


def _sc_combine_body(y_hbm, d0_hbm, d1_hbm, a0_hbm, a1_hbm,
                     d0_v, d1_v, r0, r1, r2, r3,
                     si0, si1, si2, si3, so0, so1, so2, so3):
    c = lax.axis_index("c")
    s = lax.axis_index("s")
    wid = s * 2 + c
    base = wid * _TPW

    pltpu.sync_copy(d0_hbm.at[pl.ds(wid * _NCH, _NCH)], d0_v)
    pltpu.sync_copy(d1_hbm.at[pl.ds(wid * _NCH, _NCH)], d1_v)

    rows = [r0, r1, r2, r3]
    si = [si0, si1, si2, si3]
    so = [so0, so1, so2, so3]
    dv = [d0_v, d1_v]
    ah = [a0_hbm, a1_hbm]
    ntask = 2 * _NCH

    def gather(k):
        slot, ch = k & 1, k >> 1
        b = k % _NB
        return pltpu.async_copy(y_hbm.at[dv[slot].at[ch]], rows[b], si[b])

    ins = {k: gather(k) for k in range(_NB)}
    outs = {}
    for k in range(ntask):
        slot, ch = k & 1, k >> 1
        b = k % _NB
        if k >= 2 and k + 2 < ntask:
            outs[k - 2].wait()
            ins[k + 2] = gather(k + 2)
        ins[k].wait()
        outs[k] = pltpu.async_copy(
            rows[b], ah[slot].at[pl.ds(base + ch * _CH, _CH)], so[b])
    for k in range(max(0, ntask - 4), ntask):
        outs[k].wait()


def _ffn_kernel(tbl_ref, buf_ref, w1_ref, b1_ref, w2_ref, b2_ref, y_ref):
    hf = _FT // 2

    def dot(a, b):
        return jax.lax.dot_general(
            a, b, (((1,), (0,)), ((), ())),
            preferred_element_type=jnp.float32)

    xa = buf_ref[:hf, :].astype(jnp.bfloat16)
    xb = buf_ref[hf:, :].astype(jnp.bfloat16)
    w1 = w1_ref[0]
    w2 = w2_ref[0]
    ha = dot(xa, w1) + b1_ref[0]
    hb = dot(xb, w1) + b1_ref[0]
    ga = jax.nn.gelu(ha.astype(jnp.bfloat16))
    gb = jax.nn.gelu(hb.astype(jnp.bfloat16))
    y_ref[:hf, :] = dot(ga, w2) + b2_ref[0]
    y_ref[hf:, :] = dot(gb, w2) + b2_ref[0]


def _combine_kernel(x_ref, a0_ref, a1_ref, w0_ref, w1_ref, out_ref):
    w0 = w0_ref[...]
    w1 = w1_ref[...]
    out_ref[...] = (x_ref[...]
                    + jnp.where(w0 > 0, w0 * a0_ref[...], 0.0)
                    + jnp.where(w1 > 0, w1 * a1_ref[...], 0.0))


def _run_router(x, w_gate):
    wg_pad = jnp.zeros((_D, _LANES), jnp.float32).at[:, :_NUM_EXPERTS].set(
        w_gate)
    nb = _N // _TB
    return pl.pallas_call(
        _router_kernel,
        grid=(nb,),
        in_specs=[
            pl.BlockSpec((_TB, _D), lambda i: (i, 0)),
            pl.BlockSpec((_D, _LANES), lambda i: (0, 0)),
        ],
        out_specs=[
            pl.BlockSpec((_TB, 1), lambda i: (i, 0)),
            pl.BlockSpec((_TB, 1), lambda i: (i, 0)),
            pl.BlockSpec((_TB, 1), lambda i: (i, 0)),
            pl.BlockSpec((_TB, 1), lambda i: (i, 0)),
            pl.BlockSpec((_TB, 1), lambda i: (i, 0)),
            pl.BlockSpec((_TB, 1), lambda i: (i, 0)),
            pl.BlockSpec((1, _LANES), lambda i: (0, 0)),
            pl.BlockSpec((1, _LANES), lambda i: (0, 0)),
            pl.BlockSpec((1, 1), lambda i: (0, 0)),
        ],
        out_shape=[
            jax.ShapeDtypeStruct((_N, 1), jnp.int32),
            jax.ShapeDtypeStruct((_N, 1), jnp.int32),
            jax.ShapeDtypeStruct((_N, 1), jnp.int32),
            jax.ShapeDtypeStruct((_N, 1), jnp.int32),
            jax.ShapeDtypeStruct((_N, 1), jnp.float32),
            jax.ShapeDtypeStruct((_N, 1), jnp.float32),
            jax.ShapeDtypeStruct((1, _LANES), jnp.float32),
            jax.ShapeDtypeStruct((1, _LANES), jnp.int32),
            jax.ShapeDtypeStruct((1, 1), jnp.float32),
        ],
        scratch_shapes=[
            pltpu.VMEM((1, _LANES), jnp.float32),
            pltpu.VMEM((1, _LANES), jnp.float32),
            pltpu.VMEM((1, _LANES), jnp.float32),
        ],
    )(x, wg_pad)


def _run_dest(i0, i1, p0, p1, offs):
    nb = _N // _TB
    return pl.pallas_call(
        _dest_kernel,
        grid=(nb,),
        in_specs=[
            pl.BlockSpec((_TB, 1), lambda i: (i, 0)),
            pl.BlockSpec((_TB, 1), lambda i: (i, 0)),
            pl.BlockSpec((_TB, 1), lambda i: (i, 0)),
            pl.BlockSpec((_TB, 1), lambda i: (i, 0)),
            pl.BlockSpec((1, _LANES), lambda i: (0, 0)),
        ],
        out_specs=[
            pl.BlockSpec((_TB, 1), lambda i: (i, 0)),
            pl.BlockSpec((_TB, 1), lambda i: (i, 0)),
        ],
        out_shape=[
            jax.ShapeDtypeStruct((_N, 1), jnp.int32),
            jax.ShapeDtypeStruct((_N, 1), jnp.int32),
        ],
    )(i0, i1, p0, p1, offs)


_SC_SCRATCH = (
    [pltpu.VMEM((_NCH, _CH), jnp.int32)] * 2
    + [pltpu.VMEM((_CH, _D), jnp.float32)] * _NB
    + [pltpu.SemaphoreType.DMA] * (2 * _NB)
)


def _run_sc_dispatch(x, d0, d1):
    mesh = plsc.VectorSubcoreMesh(core_axis_name="c", subcore_axis_name="s")
    kfn = pl.kernel(
        _sc_dispatch_body,
        mesh=mesh,
        out_type=jax.ShapeDtypeStruct((_RBUF, _D), jnp.float32),
        scratch_types=_SC_SCRATCH,
    )
    return kfn(x, d0, d1)


def _run_sc_combine(y, d0, d1):
    mesh = plsc.VectorSubcoreMesh(core_axis_name="c", subcore_axis_name="s")
    kfn = pl.kernel(
        _sc_combine_body,
        mesh=mesh,
        out_type=[
            jax.ShapeDtypeStruct((_N, _D), jnp.float32),
            jax.ShapeDtypeStruct((_N, _D), jnp.float32),
        ],
        scratch_types=_SC_SCRATCH,
    )
    return kfn(y, d0, d1)


def _run_combine(x, a0, a1, w0, w1):
    return pl.pallas_call(
        _combine_kernel,
        grid=(_N // _TB,),
        in_specs=[
            pl.BlockSpec((_TB, _D), lambda i: (i, 0)),
            pl.BlockSpec((_TB, _D), lambda i: (i, 0)),
            pl.BlockSpec((_TB, _D), lambda i: (i, 0)),
            pl.BlockSpec((_TB, 1), lambda i: (i, 0)),
            pl.BlockSpec((_TB, 1), lambda i: (i, 0)),
        ],
        out_specs=pl.BlockSpec((_TB, _D), lambda i: (i, 0)),
        out_shape=jax.ShapeDtypeStruct((_N, _D), jnp.float32),
    )(x, a0, a1, w0, w1)


def _run_ffn(buf, tbl, W1h, b1, W2h, b2):
    return pl.pallas_call(
        _ffn_kernel,
        grid_spec=pltpu.PrefetchScalarGridSpec(
            num_scalar_prefetch=1,
            grid=(_NT - 1,),
            in_specs=[
                pl.BlockSpec((_FT, _D), lambda i, tbl: (i, 0)),
                pl.BlockSpec((1, _D, _D), lambda i, tbl: (tbl[i], 0, 0)),
                pl.BlockSpec((1, 1, _D), lambda i, tbl: (tbl[i], 0, 0)),
                pl.BlockSpec((1, _D, _D), lambda i, tbl: (tbl[i], 0, 0)),
                pl.BlockSpec((1, 1, _D), lambda i, tbl: (tbl[i], 0, 0)),
            ],
            out_specs=pl.BlockSpec((_FT, _D), lambda i, tbl: (i, 0)),
        ),
        out_shape=jax.ShapeDtypeStruct((_RBUF, _D), jnp.float32),
    )(tbl, buf, W1h, b1, W2h, b2)


@jax.jit
def kernel(x, w_gate, W1, b1, W2, b2):
    i0, i1, p0, p1, w0, w1, offs, tbl, loss = _run_router(x, w_gate)
    d0, d1 = _run_dest(i0, i1, p0, p1, offs)

    d0r = jnp.reshape(d0, (_N // _CH, _CH))
    d1r = jnp.reshape(d1, (_N // _CH, _CH))
    tbl_flat = jnp.reshape(tbl, (_LANES,))

    buf = _run_sc_dispatch(x, d0r, d1r)
    y = _run_ffn(buf, tbl_flat, W1.astype(jnp.bfloat16),
                 b1.reshape(_NUM_EXPERTS, 1, _D),
                 W2.astype(jnp.bfloat16),
                 b2.reshape(_NUM_EXPERTS, 1, _D))
    a0, a1 = _run_sc_combine(y, d0r, d1r)
    out = _run_combine(x, a0, a1, w0, w1)
    return (out, jnp.reshape(loss, ()))
